# Initial kernel scaffold; baseline (speedup 1.0000x reference)
import jax
import jax.numpy as jnp
from jax.experimental import pallas as pl


def kernel(user_feat, item_feat, click_src, click_dst, cb_src, cb_dst, W1c, b1c, W1cb, b1cb, Ws2c, Wn2c, b2c, Ws2cb, Wn2cb, b2cb, Wm, bm):
    raise NotImplementedError("write your pallas kernel here")


# speedup vs baseline: 10.6463x; 10.6463x over previous
import functools
import jax
import jax.numpy as jnp
from jax import lax
from jax.experimental import pallas as pl
from jax.experimental.pallas import tpu as pltpu
from jax.experimental.pallas import tpu_sc as plsc

N_USER = 10000
N_ITEM = 10000
NN = N_USER + N_ITEM
E = 320000
D = 128

NC = 2
NS = 16
L = 16

CHUNK = 128
SUP = 16
ACC_PAD = 10240
DEG_PAD = 20480

def _sc_kernel(out_type, scratch_types, **kw):
    def deco(f):
        @functools.lru_cache(maxsize=None)
        def build():
            mesh = plsc.VectorSubcoreMesh(core_axis_name="c", subcore_axis_name="s")
            return pl.kernel(f, out_type=out_type, mesh=mesh,
                             scratch_types=scratch_types, **kw)

        @functools.wraps(f)
        def call(*args):
            return build()(*args)

        return call
    return deco


def _wid():
    return lax.axis_index("s"), lax.axis_index("c")


EPT_DEG = 655360 // NS
RPT_DEG = EPT_DEG // CHUNK


@_sc_kernel(
    out_type=jax.ShapeDtypeStruct((2, DEG_PAD), jnp.float32),
    scratch_types=[
        pltpu.VMEM_SHARED((DEG_PAD,), jnp.float32),
        pltpu.VMEM((SUP, CHUNK), jnp.int32),
        pltpu.VMEM((CHUNK,), jnp.float32),
        pltpu.VMEM((1280,), jnp.float32),
        pltpu.SemaphoreType.DMA,
    ],
)
def _k1_degrees(idx_hbm, out_hbm, acc, idxb, ones_v, zbuf, sem):
    s, c = _wid()

    def fill1(i, _):
        ones_v[pl.ds(i * L, L)] = jnp.ones((L,), jnp.float32)
        return 0

    lax.fori_loop(0, CHUNK // L, fill1, 0)

    def fillz(i, _):
        zbuf[pl.ds(i * L, L)] = jnp.zeros((L,), jnp.float32)
        return 0

    lax.fori_loop(0, 1280 // L, fillz, 0)

    pltpu.sync_copy(zbuf, acc.at[pl.ds(s * 1280, 1280)])
    plsc.subcore_barrier()

    def sup_body(g, _):
        r0 = s * RPT_DEG + g * SUP
        pltpu.sync_copy(idx_hbm.at[c, pl.ds(r0, SUP), :], idxb)
        cps = []
        for j in range(SUP):
            cps.append(pltpu.async_copy(ones_v, acc.at[idxb.at[j]], sem, add=True))
        for cp in cps:
            cp.wait()
        return 0

    lax.fori_loop(0, RPT_DEG // SUP, sup_body, 0)
    plsc.subcore_barrier()

    pltpu.sync_copy(acc.at[pl.ds(s * 1280, 1280)], zbuf)
    pltpu.sync_copy(zbuf, out_hbm.at[c, pl.ds(s * 1280, 1280)])


EPT_SPMM = 327680 // NS
RPT_SPMM = EPT_SPMM // CHUNK


@_sc_kernel(
    out_type=jax.ShapeDtypeStruct((NN, D), jnp.float32),
    scratch_types=[
        pltpu.VMEM_SHARED((ACC_PAD, D), jnp.float32),
        pltpu.VMEM((SUP, CHUNK), jnp.int32),
        pltpu.VMEM((SUP, CHUNK), jnp.int32),
        pltpu.VMEM((CHUNK, D), jnp.float32),
        pltpu.VMEM((CHUNK, D), jnp.float32),
        pltpu.SemaphoreType.DMA,
        pltpu.SemaphoreType.DMA,
    ],
)
def _k2_spmm(table_hbm, sidx_hbm, didx_hbm, out_hbm,
             acc, sidxb, didxb, rows_a, rows_b, sem_g, sem_s):
    s, c = _wid()

    def fillz(i, _):
        rows_a[i // 8, pl.ds((i % 8) * L, L)] = jnp.zeros((L,), jnp.float32)
        return 0

    lax.fori_loop(0, CHUNK * D // L, fillz, 0)
    for k in range(ACC_PAD // NS // CHUNK):
        pltpu.sync_copy(rows_a, acc.at[pl.ds(s * (ACC_PAD // NS) + k * CHUNK, CHUNK), :])
    plsc.subcore_barrier()

    nsup = RPT_SPMM // SUP

    def sup_body(g, _):
        r0 = s * RPT_SPMM + g * SUP
        pltpu.sync_copy(sidx_hbm.at[c, pl.ds(r0, SUP), :], sidxb)
        pltpu.sync_copy(didx_hbm.at[c, pl.ds(r0, SUP), :], didxb)
        gp = pltpu.async_copy(table_hbm.at[sidxb.at[0]], rows_a, sem_g)
        for j in range(SUP):
            cur, nxt = (rows_a, rows_b) if j % 2 == 0 else (rows_b, rows_a)
            gn = None
            if j + 1 < SUP:
                gn = pltpu.async_copy(table_hbm.at[sidxb.at[j + 1]], nxt, sem_g)
            gp.wait()
            pltpu.async_copy(cur, acc.at[didxb.at[j]], sem_s, add=True).wait()
            gp = gn
        return 0

    lax.fori_loop(0, nsup, sup_body, 0)
    plsc.subcore_barrier()

    for k in range(5):
        r = (s + NS * k) * CHUNK

        @pl.when(r < 9984)
        def _():
            pltpu.sync_copy(acc.at[pl.ds(r, CHUNK), :], rows_a)
            pltpu.sync_copy(rows_a, out_hbm.at[pl.ds(c * N_USER + r, CHUNK), :])

    @pl.when(s == 15)
    def _():
        pltpu.sync_copy(acc.at[pl.ds(9984, 16), :], rows_b.at[pl.ds(0, 16), :])
        pltpu.sync_copy(rows_b.at[pl.ds(0, 16), :],
                        out_hbm.at[pl.ds(c * N_USER + 9984, 16), :])


@_sc_kernel(
    out_type=jax.ShapeDtypeStruct((NN,), jnp.float32),
    scratch_types=[
        pltpu.VMEM_SHARED((ACC_PAD,), jnp.float32),
        pltpu.VMEM((SUP, CHUNK), jnp.int32),
        pltpu.VMEM((SUP, CHUNK), jnp.int32),
        pltpu.VMEM((SUP, CHUNK), jnp.float32),
        pltpu.VMEM((1000,), jnp.float32),
        pltpu.SemaphoreType.DMA,
        pltpu.SemaphoreType.DMA,
    ],
)
def _k3_scalar_spmm(nt_hbm, sidx_hbm, didx_hbm, out_hbm,
                    acc, sidxb, didxb, vals, zbuf, sem_g, sem_s):
    s, c = _wid()

    def fillz(i, _):
        zbuf[pl.ds(i * L, L)] = jnp.zeros((L,), jnp.float32)
        return 0

    lax.fori_loop(0, 1000 // L, fillz, 0)
    zbuf[pl.ds(984, L)] = jnp.zeros((L,), jnp.float32)
    pltpu.sync_copy(zbuf.at[pl.ds(0, 640)], acc.at[pl.ds(s * 640, 640)])
    plsc.subcore_barrier()

    nsup = RPT_SPMM // SUP

    def sup_body(g, _):
        r0 = s * RPT_SPMM + g * SUP
        pltpu.sync_copy(sidx_hbm.at[c, pl.ds(r0, SUP), :], sidxb)
        pltpu.sync_copy(didx_hbm.at[c, pl.ds(r0, SUP), :], didxb)
        gps = [pltpu.async_copy(nt_hbm.at[sidxb.at[j]], vals.at[j], sem_g)
               for j in range(SUP)]
        sps = []
        for j in range(SUP):
            gps[j].wait()
            sps.append(pltpu.async_copy(vals.at[j], acc.at[didxb.at[j]], sem_s,
                                        add=True))
        for cp in sps:
            cp.wait()
        return 0

    lax.fori_loop(0, nsup, sup_body, 0)
    plsc.subcore_barrier()

    @pl.when(s < 10)
    def _():
        pltpu.sync_copy(acc.at[pl.ds(s * 1000, 1000)], zbuf)
        pltpu.sync_copy(zbuf, out_hbm.at[pl.ds(c * N_USER + s * 1000, 1000)])


EPT_PRED = E // (NC * NS)


@_sc_kernel(
    out_type=jax.ShapeDtypeStruct((E,), jnp.float32),
    scratch_types=[
        pltpu.VMEM((NN,), jnp.float32),
        pltpu.VMEM((EPT_PRED,), jnp.int32),
        pltpu.VMEM((EPT_PRED,), jnp.int32),
        pltpu.VMEM((EPT_PRED,), jnp.float32),
    ],
    compiler_params=pltpu.CompilerParams(needs_layout_passes=False),
)
def _k4_score(stab_hbm, src_hbm, dst_hbm, out_hbm, stab_v, sidx_v, didx_v, out_v):
    s, c = _wid()
    wid = s * NC + c
    base = wid * EPT_PRED
    pltpu.sync_copy(stab_hbm, stab_v)
    pltpu.sync_copy(src_hbm.at[pl.ds(base, EPT_PRED)], sidx_v)
    pltpu.sync_copy(dst_hbm.at[pl.ds(base, EPT_PRED)], didx_v)

    def body(i, _):
        ii = pl.ds(i * L, L)
        a = plsc.load_gather(stab_v, [sidx_v[ii]])
        b = plsc.load_gather(stab_v, [didx_v[ii] + N_USER])
        g = a + b
        out_v[ii] = 1.0 / (1.0 + jnp.exp(-g))
        return 0

    lax.fori_loop(0, EPT_PRED // L, body, 0)
    pltpu.sync_copy(out_v, out_hbm.at[pl.ds(base, EPT_PRED)])


TC_B = 2000


def _tc1_body(feat_ref, deg_ref, wu_ref, wi_ref, out_ref):
    pid = pl.program_id(0)
    rs = lax.rsqrt(jnp.maximum(deg_ref[...], 1.0))
    x = feat_ref[...] * rs
    w = jnp.where(pid < 5, wu_ref[...], wi_ref[...])
    out_ref[...] = jnp.dot(x, w, preferred_element_type=jnp.float32)


def _tc1(feat_all, dego2d, W1c, W1cb):
    grid = NN // TC_B
    return pl.pallas_call(
        _tc1_body,
        grid=(grid,),
        in_specs=[
            pl.BlockSpec((TC_B, D), lambda i: (i, 0)),
            pl.BlockSpec((TC_B, 1), lambda i: (i, 0)),
            pl.BlockSpec((D, D), lambda i: (0, 0)),
            pl.BlockSpec((D, D), lambda i: (0, 0)),
        ],
        out_specs=pl.BlockSpec((TC_B, D), lambda i: (i, 0)),
        out_shape=jax.ShapeDtypeStruct((NN, D), jnp.float32),
    )(feat_all, dego2d, W1c, W1cb)


def _tc2_body(agg_ref, din_ref, b1c_ref, b1cb_ref, ws2c_ref, wn2c_ref,
              ws2cb_ref, wn2cb_ref, b2c_ref, b2cb_ref, wm_ref, bm_ref,
              nt_ref, self_ref, invd_ref):
    pid = pl.program_id(0)
    is_user = pid < 5
    d = jnp.maximum(din_ref[...], 1.0)
    bias = jnp.where(is_user, b1cb_ref[...], b1c_ref[...])
    h = jax.nn.relu(agg_ref[...] * lax.rsqrt(d) + bias)
    wmu = wm_ref[0:D, :]
    wmv = wm_ref[D:2 * D, :]
    wn = jnp.where(is_user,
                   jnp.dot(wn2c_ref[...], wmv, preferred_element_type=jnp.float32),
                   jnp.dot(wn2cb_ref[...], wmu, preferred_element_type=jnp.float32))
    ws = jnp.where(is_user,
                   jnp.dot(ws2cb_ref[...], wmu, preferred_element_type=jnp.float32),
                   jnp.dot(ws2c_ref[...], wmv, preferred_element_type=jnp.float32))
    cc = jnp.where(is_user,
                   jnp.dot(b2cb_ref[...], wmu, preferred_element_type=jnp.float32),
                   jnp.dot(b2c_ref[...], wmv, preferred_element_type=jnp.float32)
                   + bm_ref[...])
    nt_ref[...] = jnp.dot(h, wn, preferred_element_type=jnp.float32)
    self_ref[...] = jnp.dot(h, ws, preferred_element_type=jnp.float32) + cc
    invd_ref[...] = 1.0 / d


def _tc2(agg1, din2d, b1c, b1cb, Ws2c, Wn2c, Ws2cb, Wn2cb, b2c, b2cb, Wm, bm):
    grid = NN // TC_B
    full = lambda i: (0, 0)
    return pl.pallas_call(
        _tc2_body,
        grid=(grid,),
        in_specs=[
            pl.BlockSpec((TC_B, D), lambda i: (i, 0)),
            pl.BlockSpec((TC_B, 1), lambda i: (i, 0)),
            pl.BlockSpec((1, D), full),
            pl.BlockSpec((1, D), full),
            pl.BlockSpec((D, D), full),
            pl.BlockSpec((D, D), full),
            pl.BlockSpec((D, D), full),
            pl.BlockSpec((D, D), full),
            pl.BlockSpec((1, D), full),
            pl.BlockSpec((1, D), full),
            pl.BlockSpec((2 * D, 1), full),
            pl.BlockSpec((1, 1), full),
        ],
        out_specs=[
            pl.BlockSpec((TC_B, 1), lambda i: (i, 0)),
            pl.BlockSpec((TC_B, 1), lambda i: (i, 0)),
            pl.BlockSpec((TC_B, 1), lambda i: (i, 0)),
        ],
        out_shape=[
            jax.ShapeDtypeStruct((NN, 1), jnp.float32),
            jax.ShapeDtypeStruct((NN, 1), jnp.float32),
            jax.ShapeDtypeStruct((NN, 1), jnp.float32),
        ],
    )(agg1, din2d, b1c, b1cb, Ws2c, Wn2c, Ws2cb, Wn2cb, b2c, b2cb, Wm, bm)


def _tc3_body(self_ref, seg_ref, invd_ref, out_ref):
    out_ref[...] = self_ref[...] + seg_ref[...] * invd_ref[...]


def _tc3(self2d, seg2d, invd2d):
    return pl.pallas_call(
        _tc3_body,
        out_shape=jax.ShapeDtypeStruct((NN, 1), jnp.float32),
    )(self2d, seg2d, invd2d)


def kernel(user_feat, item_feat, click_src, click_dst, cb_src, cb_dst,
           W1c, b1c, W1cb, b1cb, Ws2c, Wn2c, b2c, Ws2cb, Wn2cb, b2cb, Wm, bm):
    i32 = jnp.int32
    click_src = click_src.astype(i32)
    click_dst = click_dst.astype(i32)
    cb_src = cb_src.astype(i32)
    cb_dst = cb_dst.astype(i32)

    npad_deg = NS * EPT_DEG - 2 * E
    padd = (jnp.arange(npad_deg, dtype=i32) % 480) + NN
    deg_idx = jnp.stack([
        jnp.concatenate([click_src, cb_dst + N_USER, padd]),
        jnp.concatenate([click_dst, cb_src + N_USER, padd]),
    ]).reshape(2, NS * RPT_DEG, CHUNK)

    npad = NS * EPT_SPMM - E
    pad_src = jnp.arange(npad, dtype=i32) % NN
    pad_dst = (jnp.arange(npad, dtype=i32) % 240) + N_USER
    src_idx = jnp.stack([
        jnp.concatenate([cb_src + N_USER, pad_src]),
        jnp.concatenate([click_src, pad_src]),
    ]).reshape(2, NS * RPT_SPMM, CHUNK)
    dst_idx = jnp.stack([
        jnp.concatenate([cb_dst, pad_dst]),
        jnp.concatenate([click_dst, pad_dst]),
    ]).reshape(2, NS * RPT_SPMM, CHUNK)

    feat_all = jnp.concatenate([user_feat, item_feat], axis=0)

    degs = _k1_degrees(deg_idx)
    dego2d = jnp.concatenate([degs[0, :N_USER], degs[1, N_USER:NN]]).reshape(NN, 1)
    din2d = jnp.concatenate([degs[0, N_USER:NN], degs[1, :N_USER]]).reshape(NN, 1)

    table1 = _tc1(feat_all, dego2d, W1c, W1cb)
    agg1 = _k2_spmm(table1, src_idx, dst_idx)

    nt2d, self2d, invd2d = _tc2(agg1, din2d,
                                b1c.reshape(1, D), b1cb.reshape(1, D),
                                Ws2c, Wn2c, Ws2cb, Wn2cb,
                                b2c.reshape(1, D), b2cb.reshape(1, D),
                                Wm, bm.reshape(1, 1))

    seg = _k3_scalar_spmm(nt2d.reshape(NN), src_idx, dst_idx)

    stab = _tc3(self2d, seg.reshape(NN, 1), invd2d)

    score = _k4_score(stab.reshape(NN), click_src, click_dst)
    return score.reshape(E, 1)


# speedup vs baseline: 10.7964x; 1.0141x over previous
import functools
import jax
import jax.numpy as jnp
from jax import lax
from jax.experimental import pallas as pl
from jax.experimental.pallas import tpu as pltpu
from jax.experimental.pallas import tpu_sc as plsc

N_USER = 10000
N_ITEM = 10000
NN = N_USER + N_ITEM
E = 320000
D = 128

NC = 2
NS = 16
L = 16

CHUNK = 128
SUP = 16
ACC_PAD = 10240
DEG_PAD = 20480

def _sc_kernel(out_type, scratch_types, **kw):
    def deco(f):
        @functools.lru_cache(maxsize=None)
        def build():
            mesh = plsc.VectorSubcoreMesh(core_axis_name="c", subcore_axis_name="s")
            return pl.kernel(f, out_type=out_type, mesh=mesh,
                             scratch_types=scratch_types, **kw)

        @functools.wraps(f)
        def call(*args):
            return build()(*args)

        return call
    return deco


def _wid():
    return lax.axis_index("s"), lax.axis_index("c")


EPT_DEG = 655360 // NS
RPT_DEG = EPT_DEG // CHUNK


@_sc_kernel(
    out_type=jax.ShapeDtypeStruct((2, DEG_PAD), jnp.float32),
    scratch_types=[
        pltpu.VMEM_SHARED((DEG_PAD,), jnp.float32),
        pltpu.VMEM((SUP, CHUNK), jnp.int32),
        pltpu.VMEM((CHUNK,), jnp.float32),
        pltpu.VMEM((1280,), jnp.float32),
        pltpu.SemaphoreType.DMA,
    ],
)
def _k1_degrees(idx_hbm, out_hbm, acc, idxb, ones_v, zbuf, sem):
    s, c = _wid()

    def fill1(i, _):
        ones_v[pl.ds(i * L, L)] = jnp.ones((L,), jnp.float32)
        return 0

    lax.fori_loop(0, CHUNK // L, fill1, 0)

    def fillz(i, _):
        zbuf[pl.ds(i * L, L)] = jnp.zeros((L,), jnp.float32)
        return 0

    lax.fori_loop(0, 1280 // L, fillz, 0)

    pltpu.sync_copy(zbuf, acc.at[pl.ds(s * 1280, 1280)])
    plsc.subcore_barrier()

    def sup_body(g, _):
        r0 = s * RPT_DEG + g * SUP
        pltpu.sync_copy(idx_hbm.at[c, pl.ds(r0, SUP), :], idxb)
        cps = []
        for j in range(SUP):
            cps.append(pltpu.async_copy(ones_v, acc.at[idxb.at[j]], sem, add=True))
        for cp in cps:
            cp.wait()
        return 0

    lax.fori_loop(0, RPT_DEG // SUP, sup_body, 0)
    plsc.subcore_barrier()

    pltpu.sync_copy(acc.at[pl.ds(s * 1280, 1280)], zbuf)
    pltpu.sync_copy(zbuf, out_hbm.at[c, pl.ds(s * 1280, 1280)])


EPT_SPMM = 327680 // NS
RPT_SPMM = EPT_SPMM // CHUNK


@_sc_kernel(
    out_type=jax.ShapeDtypeStruct((NN, D), jnp.float32),
    scratch_types=[
        pltpu.VMEM_SHARED((ACC_PAD, D), jnp.float32),
        pltpu.VMEM((SUP, CHUNK), jnp.int32),
        pltpu.VMEM((SUP, CHUNK), jnp.int32),
        pltpu.VMEM((2, CHUNK, D), jnp.float32),
        pltpu.SemaphoreType.DMA,
        pltpu.SemaphoreType.DMA,
    ],
)
def _k2_spmm(table_hbm, sidx_hbm, didx_hbm, out_hbm,
             acc, sidxb, didxb, rows, sem_g, sem_s):
    s, c = _wid()
    NB = 2

    def fillz(i, _):
        rows[0, i // 8, pl.ds((i % 8) * L, L)] = jnp.zeros((L,), jnp.float32)
        return 0

    lax.fori_loop(0, CHUNK * D // L, fillz, 0)
    for k in range(ACC_PAD // NS // CHUNK):
        pltpu.sync_copy(rows.at[0], acc.at[pl.ds(s * (ACC_PAD // NS) + k * CHUNK, CHUNK), :])
    plsc.subcore_barrier()

    nsup = RPT_SPMM // SUP

    def sup_body(g, _):
        r0 = s * RPT_SPMM + g * SUP
        pltpu.sync_copy(sidx_hbm.at[c, pl.ds(r0, SUP), :], sidxb)
        pltpu.sync_copy(didx_hbm.at[c, pl.ds(r0, SUP), :], didxb)
        LAG = 1
        gd = [None] * SUP
        sd = [None] * SUP
        for t in range(SUP + LAG):
            if t < SUP:
                if t >= NB:
                    sd[t - NB].wait()
                gd[t] = pltpu.async_copy(table_hbm.at[sidxb.at[t]],
                                         rows.at[t % NB], sem_g)
            if t >= LAG and t - LAG < SUP:
                j = t - LAG
                gd[j].wait()
                sd[j] = pltpu.async_copy(rows.at[j % NB], acc.at[didxb.at[j]],
                                         sem_s, add=True)
        for j in range(SUP - NB, SUP):
            sd[j].wait()
        return 0

    lax.fori_loop(0, nsup, sup_body, 0)
    plsc.subcore_barrier()

    for k in range(5):
        r = (s + NS * k) * CHUNK

        @pl.when(r < 9984)
        def _():
            pltpu.sync_copy(acc.at[pl.ds(r, CHUNK), :], rows.at[0])
            pltpu.sync_copy(rows.at[0], out_hbm.at[pl.ds(c * N_USER + r, CHUNK), :])

    @pl.when(s == 15)
    def _():
        pltpu.sync_copy(acc.at[pl.ds(9984, 16), :], rows.at[1, pl.ds(0, 16), :])
        pltpu.sync_copy(rows.at[1, pl.ds(0, 16), :],
                        out_hbm.at[pl.ds(c * N_USER + 9984, 16), :])


@_sc_kernel(
    out_type=jax.ShapeDtypeStruct((NN,), jnp.float32),
    scratch_types=[
        pltpu.VMEM_SHARED((ACC_PAD,), jnp.float32),
        pltpu.VMEM((SUP, CHUNK), jnp.int32),
        pltpu.VMEM((SUP, CHUNK), jnp.int32),
        pltpu.VMEM((SUP, CHUNK), jnp.float32),
        pltpu.VMEM((2000,), jnp.float32),
        pltpu.VMEM((2000,), jnp.float32),
        pltpu.VMEM((2000,), jnp.float32),
        pltpu.SemaphoreType.DMA,
        pltpu.SemaphoreType.DMA,
    ],
)
def _k3_scalar_spmm(nt_hbm, sidx_hbm, didx_hbm, self_hbm, invd_hbm, out_hbm,
                    acc, sidxb, didxb, vals, segb, selfb, invb, sem_g, sem_s):
    s, c = _wid()

    def fillz(i, _):
        segb[pl.ds(i * L, L)] = jnp.zeros((L,), jnp.float32)
        return 0

    lax.fori_loop(0, 2000 // L, fillz, 0)
    pltpu.sync_copy(segb.at[pl.ds(0, 640)], acc.at[pl.ds(s * 640, 640)])
    plsc.subcore_barrier()

    nsup = RPT_SPMM // SUP

    def sup_body(g, _):
        r0 = s * RPT_SPMM + g * SUP
        pltpu.sync_copy(sidx_hbm.at[c, pl.ds(r0, SUP), :], sidxb)
        pltpu.sync_copy(didx_hbm.at[c, pl.ds(r0, SUP), :], didxb)
        gps = [pltpu.async_copy(nt_hbm.at[sidxb.at[j]], vals.at[j], sem_g)
               for j in range(SUP)]
        sps = []
        for j in range(SUP):
            gps[j].wait()
            sps.append(pltpu.async_copy(vals.at[j], acc.at[didxb.at[j]], sem_s,
                                        add=True))
        for cp in sps:
            cp.wait()
        return 0

    lax.fori_loop(0, nsup, sup_body, 0)
    plsc.subcore_barrier()

    @pl.when(s < 5)
    def _():
        base = c * N_USER + s * 2000
        pltpu.sync_copy(acc.at[pl.ds(s * 2000, 2000)], segb)
        pltpu.sync_copy(self_hbm.at[pl.ds(base, 2000)], selfb)
        pltpu.sync_copy(invd_hbm.at[pl.ds(base, 2000)], invb)

        def fuse(i, _):
            ii = pl.ds(i * L, L)
            segb[ii] = selfb[ii] + segb[ii] * invb[ii]
            return 0

        lax.fori_loop(0, 2000 // L, fuse, 0)
        pltpu.sync_copy(segb, out_hbm.at[pl.ds(base, 2000)])


EPT_PRED = E // (NC * NS)


@_sc_kernel(
    out_type=jax.ShapeDtypeStruct((E,), jnp.float32),
    scratch_types=[
        pltpu.VMEM((NN,), jnp.float32),
        pltpu.VMEM((EPT_PRED,), jnp.int32),
        pltpu.VMEM((EPT_PRED,), jnp.int32),
        pltpu.VMEM((EPT_PRED,), jnp.float32),
    ],
    compiler_params=pltpu.CompilerParams(needs_layout_passes=False),
)
def _k4_score(stab_hbm, src_hbm, dst_hbm, out_hbm, stab_v, sidx_v, didx_v, out_v):
    s, c = _wid()
    wid = s * NC + c
    base = wid * EPT_PRED
    pltpu.sync_copy(stab_hbm, stab_v)
    pltpu.sync_copy(src_hbm.at[pl.ds(base, EPT_PRED)], sidx_v)
    pltpu.sync_copy(dst_hbm.at[pl.ds(base, EPT_PRED)], didx_v)

    def body(i, _):
        ii = pl.ds(i * L, L)
        a = plsc.load_gather(stab_v, [sidx_v[ii]])
        b = plsc.load_gather(stab_v, [didx_v[ii] + N_USER])
        g = a + b
        out_v[ii] = 1.0 / (1.0 + jnp.exp(-g))
        return 0

    lax.fori_loop(0, EPT_PRED // L, body, 0)
    pltpu.sync_copy(out_v, out_hbm.at[pl.ds(base, EPT_PRED)])


TC_B = 2000


def _tc1_body(feat_ref, deg_ref, wu_ref, wi_ref, out_ref):
    pid = pl.program_id(0)
    rs = lax.rsqrt(jnp.maximum(deg_ref[...], 1.0))
    x = feat_ref[...] * rs
    w = jnp.where(pid < 5, wu_ref[...], wi_ref[...])
    out_ref[...] = jnp.dot(x, w, preferred_element_type=jnp.float32)


def _tc1(feat_all, dego2d, W1c, W1cb):
    grid = NN // TC_B
    return pl.pallas_call(
        _tc1_body,
        grid=(grid,),
        in_specs=[
            pl.BlockSpec((TC_B, D), lambda i: (i, 0)),
            pl.BlockSpec((TC_B, 1), lambda i: (i, 0)),
            pl.BlockSpec((D, D), lambda i: (0, 0)),
            pl.BlockSpec((D, D), lambda i: (0, 0)),
        ],
        out_specs=pl.BlockSpec((TC_B, D), lambda i: (i, 0)),
        out_shape=jax.ShapeDtypeStruct((NN, D), jnp.float32),
    )(feat_all, dego2d, W1c, W1cb)


def _tc2_body(agg_ref, din_ref, b1c_ref, b1cb_ref, ws2c_ref, wn2c_ref,
              ws2cb_ref, wn2cb_ref, b2c_ref, b2cb_ref, wm_ref, bm_ref,
              nt_ref, self_ref, invd_ref):
    pid = pl.program_id(0)
    is_user = pid < 5
    d = jnp.maximum(din_ref[...], 1.0)
    bias = jnp.where(is_user, b1cb_ref[...], b1c_ref[...])
    h = jax.nn.relu(agg_ref[...] * lax.rsqrt(d) + bias)
    wmu = wm_ref[0:D, :]
    wmv = wm_ref[D:2 * D, :]
    wn = jnp.where(is_user,
                   jnp.dot(wn2c_ref[...], wmv, preferred_element_type=jnp.float32),
                   jnp.dot(wn2cb_ref[...], wmu, preferred_element_type=jnp.float32))
    ws = jnp.where(is_user,
                   jnp.dot(ws2cb_ref[...], wmu, preferred_element_type=jnp.float32),
                   jnp.dot(ws2c_ref[...], wmv, preferred_element_type=jnp.float32))
    cc = jnp.where(is_user,
                   jnp.dot(b2cb_ref[...], wmu, preferred_element_type=jnp.float32),
                   jnp.dot(b2c_ref[...], wmv, preferred_element_type=jnp.float32)
                   + bm_ref[...])
    nt_ref[...] = jnp.dot(h, wn, preferred_element_type=jnp.float32)
    self_ref[...] = jnp.dot(h, ws, preferred_element_type=jnp.float32) + cc
    invd_ref[...] = 1.0 / d


def _tc2(agg1, din2d, b1c, b1cb, Ws2c, Wn2c, Ws2cb, Wn2cb, b2c, b2cb, Wm, bm):
    grid = NN // TC_B
    full = lambda i: (0, 0)
    return pl.pallas_call(
        _tc2_body,
        grid=(grid,),
        in_specs=[
            pl.BlockSpec((TC_B, D), lambda i: (i, 0)),
            pl.BlockSpec((TC_B, 1), lambda i: (i, 0)),
            pl.BlockSpec((1, D), full),
            pl.BlockSpec((1, D), full),
            pl.BlockSpec((D, D), full),
            pl.BlockSpec((D, D), full),
            pl.BlockSpec((D, D), full),
            pl.BlockSpec((D, D), full),
            pl.BlockSpec((1, D), full),
            pl.BlockSpec((1, D), full),
            pl.BlockSpec((2 * D, 1), full),
            pl.BlockSpec((1, 1), full),
        ],
        out_specs=[
            pl.BlockSpec((TC_B, 1), lambda i: (i, 0)),
            pl.BlockSpec((TC_B, 1), lambda i: (i, 0)),
            pl.BlockSpec((TC_B, 1), lambda i: (i, 0)),
        ],
        out_shape=[
            jax.ShapeDtypeStruct((NN, 1), jnp.float32),
            jax.ShapeDtypeStruct((NN, 1), jnp.float32),
            jax.ShapeDtypeStruct((NN, 1), jnp.float32),
        ],
    )(agg1, din2d, b1c, b1cb, Ws2c, Wn2c, Ws2cb, Wn2cb, b2c, b2cb, Wm, bm)


def kernel(user_feat, item_feat, click_src, click_dst, cb_src, cb_dst,
           W1c, b1c, W1cb, b1cb, Ws2c, Wn2c, b2c, Ws2cb, Wn2cb, b2cb, Wm, bm):
    i32 = jnp.int32
    click_src = click_src.astype(i32)
    click_dst = click_dst.astype(i32)
    cb_src = cb_src.astype(i32)
    cb_dst = cb_dst.astype(i32)

    npad_deg = NS * EPT_DEG - 2 * E
    padd = (jnp.arange(npad_deg, dtype=i32) % 480) + NN
    deg_idx = jnp.stack([
        jnp.concatenate([click_src, cb_dst + N_USER, padd]),
        jnp.concatenate([click_dst, cb_src + N_USER, padd]),
    ]).reshape(2, NS * RPT_DEG, CHUNK)

    npad = NS * EPT_SPMM - E
    pad_src = jnp.arange(npad, dtype=i32) % NN
    pad_dst = (jnp.arange(npad, dtype=i32) % 240) + N_USER
    src_idx = jnp.stack([
        jnp.concatenate([cb_src + N_USER, pad_src]),
        jnp.concatenate([click_src, pad_src]),
    ]).reshape(2, NS * RPT_SPMM, CHUNK)
    dst_idx = jnp.stack([
        jnp.concatenate([cb_dst, pad_dst]),
        jnp.concatenate([click_dst, pad_dst]),
    ]).reshape(2, NS * RPT_SPMM, CHUNK)

    feat_all = jnp.concatenate([user_feat, item_feat], axis=0)

    degs = _k1_degrees(deg_idx)
    dego2d = jnp.concatenate([degs[0, :N_USER], degs[1, N_USER:NN]]).reshape(NN, 1)
    din2d = jnp.concatenate([degs[0, N_USER:NN], degs[1, :N_USER]]).reshape(NN, 1)

    table1 = _tc1(feat_all, dego2d, W1c, W1cb)
    agg1 = _k2_spmm(table1, src_idx, dst_idx)

    nt2d, self2d, invd2d = _tc2(agg1, din2d,
                                b1c.reshape(1, D), b1cb.reshape(1, D),
                                Ws2c, Wn2c, Ws2cb, Wn2cb,
                                b2c.reshape(1, D), b2cb.reshape(1, D),
                                Wm, bm.reshape(1, 1))

    stab = _k3_scalar_spmm(nt2d.reshape(NN), src_idx, dst_idx,
                           self2d.reshape(NN), invd2d.reshape(NN))

    score = _k4_score(stab, click_src, click_dst)
    return score.reshape(E, 1)
